# baseline (speedup 1.0000x reference)
import functools

import jax
import jax.numpy as jnp
from jax import lax
from jax.experimental import pallas as pl
from jax.experimental.pallas import tpu as pltpu
from jax.experimental.pallas import tpu_sc as plsc

N = 10000
E = 320000
H = 166
HP = 176
FP = 336
NG = 32
QW = 48
Q3 = H - 3 * QW
NT = 16
EPT = E // NT
B = 80
NBLK = EPT // B
NACC = 10240
TPR = NACC // NT


def _pad2(w, r, c):
    return jnp.pad(w, ((0, r - w.shape[0]), (0, c - w.shape[1])))


def _padb(b, c):
    return jnp.pad(b, (0, c - b.shape[0])).reshape(1, c)


def _split48(p):
    n = p.shape[0]
    qs = [p[:, 0:48], p[:, 48:96], p[:, 96:144],
          jnp.concatenate([p[:, 144:176], jnp.zeros((n, 16), jnp.float32)], axis=1)]
    return jnp.stack(qs, axis=0)


def _cat_q(a0, a1, b0, b1, off):
    n = a0.shape[0]
    return jnp.concatenate(
        [a0[:, off:off + QW], a1[:, off:off + QW], b0[:, off:off + QW],
         b1[:, off:off + Q3], jnp.zeros((n, HP - H), jnp.float32)], axis=1)


def _col_mask():
    return (lax.broadcasted_iota(jnp.int32, (1, HP), 1) < H).astype(jnp.float32)


def _ln_masked(v, g, b):
    m = jnp.sum(v, axis=1, keepdims=True) * (1.0 / H)
    sq = jnp.sum(v * v, axis=1, keepdims=True) * (1.0 / H)
    inv = lax.rsqrt(sq - m * m + 1e-5)
    return ((v - m) * inv * g + b) * _col_mask()


def _dot(a, b):
    return jnp.dot(a, b, preferred_element_type=jnp.float32)


def _embed_body(x_ref, lp_ref, wh_ref, wp_ref, b_ref, out_ref):
    out_ref[...] = (_dot(x_ref[...], wh_ref[...])
                    + _dot(lp_ref[...], wp_ref[...]) + b_ref[...])


def _embed(x, lap, whp, wpp, b0):
    nb = 1000
    return pl.pallas_call(
        _embed_body,
        grid=(N // nb,),
        in_specs=[
            pl.BlockSpec((nb, x.shape[1]), lambda i: (i, 0)),
            pl.BlockSpec((nb, lap.shape[1]), lambda i: (i, 0)),
            pl.BlockSpec(whp.shape, lambda i: (0, 0)),
            pl.BlockSpec(wpp.shape, lambda i: (0, 0)),
            pl.BlockSpec((1, HP), lambda i: (0, 0)),
        ],
        out_specs=pl.BlockSpec((nb, HP), lambda i: (i, 0)),
        out_shape=jax.ShapeDtypeStruct((N, HP), jnp.float32),
    )(x, lap, whp, wpp, b0)


def _tables_body(h_ref, a_ref, ba_ref, bm_ref, bb_ref, v_ref, bv_ref, u_ref, bu_ref,
                 at_ref, bvt_ref, uh_ref):
    hb = h_ref[...]
    pa = _dot(hb, a_ref[...]) + ba_ref[...]
    pb = _dot(hb, bm_ref[...]) + bb_ref[...]
    pv = _dot(hb, v_ref[...]) + bv_ref[...]
    pu = _dot(hb, u_ref[...]) + bu_ref[...]
    at_ref[...] = _split48(pa)
    sb = _split48(pb)
    sv = _split48(pv)
    bvt_ref[...] = jnp.concatenate([sb, sv], axis=2)
    uh_ref[...] = pu


def _tables(h, ap, bap, bmp, bbp, vp, bvp, up, bup):
    nb = 1000
    w = pl.BlockSpec((HP, HP), lambda i: (0, 0))
    bspec = pl.BlockSpec((1, HP), lambda i: (0, 0))
    return pl.pallas_call(
        _tables_body,
        grid=(N // nb,),
        in_specs=[pl.BlockSpec((nb, HP), lambda i: (i, 0)),
                  w, bspec, w, bspec, w, bspec, w, bspec],
        out_specs=[
            pl.BlockSpec((4, nb, QW), lambda i: (0, i, 0)),
            pl.BlockSpec((4, nb, 2 * QW), lambda i: (0, i, 0)),
            pl.BlockSpec((nb, HP), lambda i: (i, 0)),
        ],
        out_shape=[
            jax.ShapeDtypeStruct((4, N, QW), jnp.float32),
            jax.ShapeDtypeStruct((4, N, 2 * QW), jnp.float32),
            jax.ShapeDtypeStruct((N, HP), jnp.float32),
        ],
    )(h, ap, bap, bmp, bbp, vp, bvp, up, bup)


def _e0_body(ea_ref, we_ref, be_ref, c_ref, bc_ref, e0_ref, ch_ref):
    eb = _dot(ea_ref[...], we_ref[...]) + be_ref[...]
    e0_ref[...] = eb
    ch_ref[...] = _split48(_dot(eb, c_ref[...]) + bc_ref[...])


def _e0(eap, wep, bep, cp, bcp):
    eb = 2000
    return pl.pallas_call(
        _e0_body,
        grid=(E // eb,),
        in_specs=[
            pl.BlockSpec((eb, eap.shape[1]), lambda i: (i, 0)),
            pl.BlockSpec(wep.shape, lambda i: (0, 0)),
            pl.BlockSpec((1, HP), lambda i: (0, 0)),
            pl.BlockSpec((HP, HP), lambda i: (0, 0)),
            pl.BlockSpec((1, HP), lambda i: (0, 0)),
        ],
        out_specs=[
            pl.BlockSpec((eb, HP), lambda i: (i, 0)),
            pl.BlockSpec((4, eb, QW), lambda i: (0, i, 0)),
        ],
        out_shape=[
            jax.ShapeDtypeStruct((E, HP), jnp.float32),
            jax.ShapeDtypeStruct((4, E, QW), jnp.float32),
        ],
    )(eap, wep, bep, cp, bcp)


def _make_eres_body(write_e):
    def body(ep_ref, ra_ref, rb_ref, g_ref, b_ref, c_ref, bc_ref, *outs):
        if write_e:
            e_ref, ch_ref = outs
        else:
            (ch_ref,) = outs
        rp = _cat_q(ra_ref[0], ra_ref[1], rb_ref[0], rb_ref[1], 0)
        v = ep_ref[...] + rp
        en = _ln_masked(v, g_ref[...], b_ref[...])
        if write_e:
            e_ref[...] = en
        ch_ref[...] = _split48(_dot(en, c_ref[...]) + bc_ref[...])
    return body


def _eres(eprev, ra, rb, gp, bp_, cp, bcp, write_e):
    eb = 2000
    out_specs = [pl.BlockSpec((4, eb, QW), lambda i: (0, i, 0))]
    out_shape = [jax.ShapeDtypeStruct((4, E, QW), jnp.float32)]
    if write_e:
        out_specs = [pl.BlockSpec((eb, HP), lambda i: (i, 0))] + out_specs
        out_shape = [jax.ShapeDtypeStruct((E, HP), jnp.float32)] + out_shape
    res = pl.pallas_call(
        _make_eres_body(write_e),
        grid=(E // eb,),
        in_specs=[
            pl.BlockSpec((eb, HP), lambda i: (i, 0)),
            pl.BlockSpec((2, eb, QW), lambda i: (0, i, 0)),
            pl.BlockSpec((2, eb, QW), lambda i: (0, i, 0)),
            pl.BlockSpec((1, HP), lambda i: (0, 0)),
            pl.BlockSpec((1, HP), lambda i: (0, 0)),
            pl.BlockSpec((HP, HP), lambda i: (0, 0)),
            pl.BlockSpec((1, HP), lambda i: (0, 0)),
        ],
        out_specs=out_specs,
        out_shape=out_shape,
    )(eprev, ra, rb, gp, bp_, cp, bcp)
    if write_e:
        return res[0], res[1]
    return None, res[0]


def _hupd_body(h_ref, uh_ref, aa_ref, ab_ref, g1_ref, b1_ref, w1_ref, bf1_ref,
               w2_ref, bf2_ref, g2_ref, b2_ref, out_ref):
    num = _cat_q(aa_ref[0], aa_ref[1], ab_ref[0], ab_ref[1], 0)
    den = _cat_q(aa_ref[0], aa_ref[1], ab_ref[0], ab_ref[1], QW)
    hn = jnp.maximum(uh_ref[...] + num / (den + 1e-6), 0.0)
    h1 = _ln_masked(h_ref[...] + hn, g1_ref[...], b1_ref[...])
    h2 = jnp.maximum(_dot(h1, w1_ref[...]) + bf1_ref[...], 0.0)
    h2 = _dot(h2, w2_ref[...]) + bf2_ref[...]
    out_ref[...] = _ln_masked(h1 + h2, g2_ref[...], b2_ref[...])


def _hupd(h, uh, aa, ab, g1p, b1p, w1p, bf1p, w2p, bf2p, g2p, b2p):
    nb = 1000
    bspec = pl.BlockSpec((1, HP), lambda i: (0, 0))
    return pl.pallas_call(
        _hupd_body,
        grid=(N // nb,),
        in_specs=[
            pl.BlockSpec((nb, HP), lambda i: (i, 0)),
            pl.BlockSpec((nb, HP), lambda i: (i, 0)),
            pl.BlockSpec((2, nb, 2 * QW), lambda i: (0, i, 0)),
            pl.BlockSpec((2, nb, 2 * QW), lambda i: (0, i, 0)),
            bspec, bspec,
            pl.BlockSpec((HP, FP), lambda i: (0, 0)),
            pl.BlockSpec((1, FP), lambda i: (0, 0)),
            pl.BlockSpec((FP, HP), lambda i: (0, 0)),
            bspec, bspec, bspec,
        ],
        out_specs=pl.BlockSpec((nb, HP), lambda i: (i, 0)),
        out_shape=jax.ShapeDtypeStruct((N, HP), jnp.float32),
    )(h, uh, aa, ab, g1p, b1p, w1p, bf1p, w2p, bf2p, g2p, b2p)


def _pool_body(h_ref, b_ref, o_ref, sums_ref, cnt_ref):
    i = pl.program_id(0)

    @pl.when(i == 0)
    def _():
        sums_ref[...] = jnp.zeros_like(sums_ref)
        cnt_ref[...] = jnp.zeros_like(cnt_ref)

    bi = b_ref[0, 0, :]
    nb = bi.shape[0]
    oh = (bi[None, :] == lax.broadcasted_iota(jnp.int32, (NG, nb), 0)).astype(jnp.float32)
    sums_ref[...] += _dot(oh, h_ref[...])
    cnt_ref[...] += _dot(oh, jnp.ones((nb, 128), jnp.float32))

    @pl.when(i == pl.num_programs(0) - 1)
    def _():
        o_ref[...] = sums_ref[...] / jnp.maximum(cnt_ref[:, :1], 1.0)


def _pool(h, batch):
    nb = 2000
    nblk = N // nb
    b3 = batch.reshape(nblk, 1, nb)
    out = pl.pallas_call(
        _pool_body,
        grid=(nblk,),
        in_specs=[
            pl.BlockSpec((nb, HP), lambda i: (i, 0)),
            pl.BlockSpec((1, 1, nb), lambda i: (i, 0, 0)),
        ],
        out_specs=pl.BlockSpec((NG, HP), lambda i: (0, 0)),
        out_shape=jax.ShapeDtypeStruct((NG, HP), jnp.float32),
        scratch_shapes=[
            pltpu.VMEM((NG, HP), jnp.float32),
            pltpu.VMEM((NG, 128), jnp.float32),
        ],
    )(h, b3)
    return out[:, :H]


@functools.lru_cache(maxsize=None)
def _make_edge_sc(write_r, k):
    mesh = plsc.VectorSubcoreMesh(core_axis_name="c", subcore_axis_name="s")
    out_type = [jax.ShapeDtypeStruct((2, NACC, 2 * QW), jnp.float32)]
    if write_r:
        out_type = [jax.ShapeDtypeStruct((2, E, QW), jnp.float32)] + out_type
    scratch_types = [
        pltpu.VMEM((B,), jnp.int32),
        pltpu.VMEM((B,), jnp.int32),
        pltpu.VMEM((B,), jnp.int32),
        pltpu.VMEM((B,), jnp.int32),
        pltpu.VMEM((B, QW), jnp.float32),
        pltpu.VMEM((B, 2 * QW), jnp.float32),
        pltpu.VMEM((B, QW), jnp.float32),
        pltpu.VMEM((B, QW), jnp.float32),
        pltpu.VMEM((B, 2 * QW), jnp.float32),
        pltpu.VMEM_SHARED((NACC, 2 * QW), jnp.float32),
        pltpu.SemaphoreType.DMA,
        pltpu.SemaphoreType.DMA,
        pltpu.SemaphoreType.DMA,
    ]

    @functools.partial(pl.kernel, mesh=mesh, out_type=out_type,
                       scratch_types=scratch_types,
                       compiler_params=pltpu.CompilerParams(
                           use_tc_tiling_on_sc=False))
    def kern(dst_hbm, src_hbm, at_hbm, bvt_hbm, ch_hbm, *refs):
        if write_r:
            r_hbm, acc_hbm = refs[0], refs[1]
            refs = refs[2:]
        else:
            acc_hbm = refs[0]
            refs = refs[1:]
        (didx, sidx, aidx, bidx, arows, bvrows, chb, rbuf, contrib,
         accum, sem_a, sem_b, sem_c) = refs
        c = lax.axis_index("c")
        s = lax.axis_index("s")
        cn = (2 * k + c) * N

        def zb(i, carry):
            for j in range(2 * QW // 16):
                contrib[i, pl.ds(j * 16, 16)] = jnp.zeros((16,), jnp.float32)
            return carry

        lax.fori_loop(0, B, zb, 0)
        for j in range(TPR // B):
            pltpu.sync_copy(contrib, accum.at[pl.ds(s * TPR + j * B, B)])
        plsc.subcore_barrier()

        base0 = s * EPT

        def blk(b, carry):
            base = base0 + b * B
            pltpu.sync_copy(dst_hbm.at[pl.ds(base, B)], didx)
            pltpu.sync_copy(src_hbm.at[pl.ds(base, B)], sidx)
            for j in range(B // 16):
                dsj = pl.ds(j * 16, 16)
                aidx[dsj] = didx[dsj] + cn
                bidx[dsj] = sidx[dsj] + cn
            cp_a = pltpu.async_copy(at_hbm.at[aidx], arows, sem_a)
            cp_b = pltpu.async_copy(bvt_hbm.at[bidx], bvrows, sem_b)
            cp_c = pltpu.async_copy(ch_hbm.at[c, pl.ds(base, B)], chb, sem_c)
            cp_a.wait()
            cp_b.wait()
            cp_c.wait()

            def edge(i, carry2):
                for kk in range(QW // 16):
                    dk = pl.ds(kk * 16, 16)
                    dk2 = pl.ds(QW + kk * 16, 16)
                    eh = arows[i, dk] + bvrows[i, dk] + chb[i, dk]
                    sg = 1.0 / (1.0 + jnp.exp(-eh))
                    if write_r:
                        rbuf[i, dk] = jnp.maximum(eh, 0.0)
                    contrib[i, dk] = sg * bvrows[i, dk2]
                    contrib[i, dk2] = sg
                return carry2

            lax.fori_loop(0, B, edge, 0)
            if write_r:
                pltpu.sync_copy(rbuf, r_hbm.at[c, pl.ds(base, B)])
            pltpu.sync_copy(contrib, accum.at[didx], add=True)
            return carry

        lax.fori_loop(0, NBLK, blk, 0)
        plsc.subcore_barrier()
        pltpu.sync_copy(accum.at[pl.ds(s * TPR, TPR)],
                        acc_hbm.at[c, pl.ds(s * TPR, TPR)])

    return kern


def _edge_sc(dst, src, at4, bvt4, ch, write_r, k):
    res = _make_edge_sc(write_r, k)(dst, src, at4, bvt4, ch[2 * k:2 * k + 2])
    if write_r:
        return res[0], res[1]
    res = res[0] if isinstance(res, (list, tuple)) else res
    return None, res


def kernel(x, edge_index, batch, lap_enc, edge_attr, Wh, bh, Wp, bp, We, be,
           A, bA, Bm, bB, C, bC, U, bU, V, bV, g1h, b1h, g1e, b1e,
           W1, bf1, W2, bf2, g2, b2):
    src = edge_index[0]
    dst = edge_index[1]

    whp = _pad2(Wh, Wh.shape[0], HP)
    wpp = _pad2(Wp, Wp.shape[0], HP)
    b0 = _padb(bh + bp, HP)
    wep = _pad2(We, 40, HP)
    bep = _padb(be, HP)
    eap = jnp.pad(edge_attr, ((0, 0), (0, 40 - edge_attr.shape[1])))

    h = _embed(x, lap_enc, whp, wpp, b0)

    eprev = None
    ra = rb = None
    for l in range(3):
        ap = _pad2(A[l], HP, HP)
        bmp = _pad2(Bm[l], HP, HP)
        cp = _pad2(C[l], HP, HP)
        up = _pad2(U[l], HP, HP)
        vp = _pad2(V[l], HP, HP)
        at, bvt, uh = _tables(h, ap, _padb(bA[l], HP), bmp, _padb(bB[l], HP),
                              vp, _padb(bV[l], HP), up, _padb(bU[l], HP))
        if l == 0:
            eprev, ch = _e0(eap, wep, bep, cp, _padb(bC[l], HP))
        else:
            eprev, ch = _eres(eprev, ra, rb, _padb(g1e[l], HP), _padb(b1e[l], HP),
                              cp, _padb(bC[l], HP), write_e=(l < 2))
        at4 = at.reshape(4 * N, QW)
        bvt4 = bvt.reshape(4 * N, 2 * QW)
        write_r = l < 2
        ra, aa = _edge_sc(dst, src, at4, bvt4, ch, write_r, 0)
        rb, ab = _edge_sc(dst, src, at4, bvt4, ch, write_r, 1)
        h = _hupd(h, uh, aa, ab, _padb(g1h[l], HP), _padb(b1h[l], HP),
                  _pad2(W1[l], HP, FP), _padb(bf1[l], FP),
                  _pad2(W2[l], FP, HP), _padb(bf2[l], HP),
                  _padb(g2[l], HP), _padb(b2[l], HP))

    return _pool(h, batch)


# speedup vs baseline: 2.1888x; 2.1888x over previous
import functools

import jax
import jax.numpy as jnp
from jax import lax
from jax.experimental import pallas as pl
from jax.experimental.pallas import tpu as pltpu
from jax.experimental.pallas import tpu_sc as plsc

N = 10000
E = 320000
H = 166
HP = 176
FP = 336
NG = 32
QW = 48
Q3 = H - 3 * QW
NT = 16
EPT = E // NT
B = 80
NBLK = EPT // B
NACC = 10240
TPR = NACC // NT


def _pad2(w, r, c):
    return jnp.pad(w, ((0, r - w.shape[0]), (0, c - w.shape[1])))


def _padb(b, c):
    return jnp.pad(b, (0, c - b.shape[0])).reshape(1, c)


def _split48(p):
    n = p.shape[0]
    qs = [p[:, 0:48], p[:, 48:96], p[:, 96:144],
          jnp.concatenate([p[:, 144:176], jnp.zeros((n, 16), jnp.float32)], axis=1)]
    return jnp.stack(qs, axis=0)


def _cat_q(a0, a1, b0, b1, off):
    n = a0.shape[0]
    return jnp.concatenate(
        [a0[:, off:off + QW], a1[:, off:off + QW], b0[:, off:off + QW],
         b1[:, off:off + Q3], jnp.zeros((n, HP - H), jnp.float32)], axis=1)


def _col_mask():
    return (lax.broadcasted_iota(jnp.int32, (1, HP), 1) < H).astype(jnp.float32)


def _ln_masked(v, g, b):
    m = jnp.sum(v, axis=1, keepdims=True) * (1.0 / H)
    sq = jnp.sum(v * v, axis=1, keepdims=True) * (1.0 / H)
    inv = lax.rsqrt(sq - m * m + 1e-5)
    return ((v - m) * inv * g + b) * _col_mask()


def _dot(a, b):
    return jnp.dot(a, b, preferred_element_type=jnp.float32)


def _embed_body(x_ref, lp_ref, wh_ref, wp_ref, b_ref, out_ref):
    out_ref[...] = (_dot(x_ref[...], wh_ref[...])
                    + _dot(lp_ref[...], wp_ref[...]) + b_ref[...])


def _embed(x, lap, whp, wpp, b0):
    nb = 1000
    return pl.pallas_call(
        _embed_body,
        grid=(N // nb,),
        in_specs=[
            pl.BlockSpec((nb, x.shape[1]), lambda i: (i, 0)),
            pl.BlockSpec((nb, lap.shape[1]), lambda i: (i, 0)),
            pl.BlockSpec(whp.shape, lambda i: (0, 0)),
            pl.BlockSpec(wpp.shape, lambda i: (0, 0)),
            pl.BlockSpec((1, HP), lambda i: (0, 0)),
        ],
        out_specs=pl.BlockSpec((nb, HP), lambda i: (i, 0)),
        out_shape=jax.ShapeDtypeStruct((N, HP), jnp.float32),
    )(x, lap, whp, wpp, b0)


def _tables_body(h_ref, a_ref, ba_ref, bm_ref, bb_ref, v_ref, bv_ref, u_ref, bu_ref,
                 at_ref, bvt_ref, uh_ref):
    hb = h_ref[...]
    pa = _dot(hb, a_ref[...]) + ba_ref[...]
    pb = _dot(hb, bm_ref[...]) + bb_ref[...]
    pv = _dot(hb, v_ref[...]) + bv_ref[...]
    pu = _dot(hb, u_ref[...]) + bu_ref[...]
    at_ref[...] = _split48(pa)
    sb = _split48(pb)
    sv = _split48(pv)
    bvt_ref[...] = jnp.concatenate([sb, sv], axis=2)
    uh_ref[...] = pu


def _tables(h, ap, bap, bmp, bbp, vp, bvp, up, bup):
    nb = 1000
    w = pl.BlockSpec((HP, HP), lambda i: (0, 0))
    bspec = pl.BlockSpec((1, HP), lambda i: (0, 0))
    return pl.pallas_call(
        _tables_body,
        grid=(N // nb,),
        in_specs=[pl.BlockSpec((nb, HP), lambda i: (i, 0)),
                  w, bspec, w, bspec, w, bspec, w, bspec],
        out_specs=[
            pl.BlockSpec((4, nb, QW), lambda i: (0, i, 0)),
            pl.BlockSpec((4, nb, 2 * QW), lambda i: (0, i, 0)),
            pl.BlockSpec((nb, HP), lambda i: (i, 0)),
        ],
        out_shape=[
            jax.ShapeDtypeStruct((4, N, QW), jnp.float32),
            jax.ShapeDtypeStruct((4, N, 2 * QW), jnp.float32),
            jax.ShapeDtypeStruct((N, HP), jnp.float32),
        ],
    )(h, ap, bap, bmp, bbp, vp, bvp, up, bup)


def _e0_body(ea_ref, we_ref, be_ref, c_ref, bc_ref, e0_ref, ch_ref):
    eb = _dot(ea_ref[...], we_ref[...]) + be_ref[...]
    e0_ref[...] = eb
    ch_ref[...] = _split48(_dot(eb, c_ref[...]) + bc_ref[...])


def _e0(eap, wep, bep, cp, bcp):
    eb = 2000
    return pl.pallas_call(
        _e0_body,
        grid=(E // eb,),
        in_specs=[
            pl.BlockSpec((eb, eap.shape[1]), lambda i: (i, 0)),
            pl.BlockSpec(wep.shape, lambda i: (0, 0)),
            pl.BlockSpec((1, HP), lambda i: (0, 0)),
            pl.BlockSpec((HP, HP), lambda i: (0, 0)),
            pl.BlockSpec((1, HP), lambda i: (0, 0)),
        ],
        out_specs=[
            pl.BlockSpec((eb, HP), lambda i: (i, 0)),
            pl.BlockSpec((4, eb, QW), lambda i: (0, i, 0)),
        ],
        out_shape=[
            jax.ShapeDtypeStruct((E, HP), jnp.float32),
            jax.ShapeDtypeStruct((4, E, QW), jnp.float32),
        ],
    )(eap, wep, bep, cp, bcp)


def _make_eres_body(write_e):
    def body(ep_ref, ra_ref, rb_ref, g_ref, b_ref, c_ref, bc_ref, *outs):
        if write_e:
            e_ref, ch_ref = outs
        else:
            (ch_ref,) = outs
        rp = _cat_q(ra_ref[0], ra_ref[1], rb_ref[0], rb_ref[1], 0)
        v = ep_ref[...] + rp
        en = _ln_masked(v, g_ref[...], b_ref[...])
        if write_e:
            e_ref[...] = en
        ch_ref[...] = _split48(_dot(en, c_ref[...]) + bc_ref[...])
    return body


def _eres(eprev, ra, rb, gp, bp_, cp, bcp, write_e):
    eb = 2000
    out_specs = [pl.BlockSpec((4, eb, QW), lambda i: (0, i, 0))]
    out_shape = [jax.ShapeDtypeStruct((4, E, QW), jnp.float32)]
    if write_e:
        out_specs = [pl.BlockSpec((eb, HP), lambda i: (i, 0))] + out_specs
        out_shape = [jax.ShapeDtypeStruct((E, HP), jnp.float32)] + out_shape
    res = pl.pallas_call(
        _make_eres_body(write_e),
        grid=(E // eb,),
        in_specs=[
            pl.BlockSpec((eb, HP), lambda i: (i, 0)),
            pl.BlockSpec((2, eb, QW), lambda i: (0, i, 0)),
            pl.BlockSpec((2, eb, QW), lambda i: (0, i, 0)),
            pl.BlockSpec((1, HP), lambda i: (0, 0)),
            pl.BlockSpec((1, HP), lambda i: (0, 0)),
            pl.BlockSpec((HP, HP), lambda i: (0, 0)),
            pl.BlockSpec((1, HP), lambda i: (0, 0)),
        ],
        out_specs=out_specs,
        out_shape=out_shape,
    )(eprev, ra, rb, gp, bp_, cp, bcp)
    if write_e:
        return res[0], res[1]
    return None, res[0]


def _hupd_body(h_ref, uh_ref, aa_ref, ab_ref, g1_ref, b1_ref, w1_ref, bf1_ref,
               w2_ref, bf2_ref, g2_ref, b2_ref, out_ref):
    num = _cat_q(aa_ref[0], aa_ref[1], ab_ref[0], ab_ref[1], 0)
    den = _cat_q(aa_ref[0], aa_ref[1], ab_ref[0], ab_ref[1], QW)
    hn = jnp.maximum(uh_ref[...] + num / (den + 1e-6), 0.0)
    h1 = _ln_masked(h_ref[...] + hn, g1_ref[...], b1_ref[...])
    h2 = jnp.maximum(_dot(h1, w1_ref[...]) + bf1_ref[...], 0.0)
    h2 = _dot(h2, w2_ref[...]) + bf2_ref[...]
    out_ref[...] = _ln_masked(h1 + h2, g2_ref[...], b2_ref[...])


def _hupd(h, uh, aa, ab, g1p, b1p, w1p, bf1p, w2p, bf2p, g2p, b2p):
    nb = 1000
    bspec = pl.BlockSpec((1, HP), lambda i: (0, 0))
    return pl.pallas_call(
        _hupd_body,
        grid=(N // nb,),
        in_specs=[
            pl.BlockSpec((nb, HP), lambda i: (i, 0)),
            pl.BlockSpec((nb, HP), lambda i: (i, 0)),
            pl.BlockSpec((2, nb, 2 * QW), lambda i: (0, i, 0)),
            pl.BlockSpec((2, nb, 2 * QW), lambda i: (0, i, 0)),
            bspec, bspec,
            pl.BlockSpec((HP, FP), lambda i: (0, 0)),
            pl.BlockSpec((1, FP), lambda i: (0, 0)),
            pl.BlockSpec((FP, HP), lambda i: (0, 0)),
            bspec, bspec, bspec,
        ],
        out_specs=pl.BlockSpec((nb, HP), lambda i: (i, 0)),
        out_shape=jax.ShapeDtypeStruct((N, HP), jnp.float32),
    )(h, uh, aa, ab, g1p, b1p, w1p, bf1p, w2p, bf2p, g2p, b2p)


def _pool_body(h_ref, b_ref, o_ref, sums_ref, cnt_ref):
    i = pl.program_id(0)

    @pl.when(i == 0)
    def _():
        sums_ref[...] = jnp.zeros_like(sums_ref)
        cnt_ref[...] = jnp.zeros_like(cnt_ref)

    bi = b_ref[0, 0, :]
    nb = bi.shape[0]
    oh = (bi[None, :] == lax.broadcasted_iota(jnp.int32, (NG, nb), 0)).astype(jnp.float32)
    sums_ref[...] += _dot(oh, h_ref[...])
    cnt_ref[...] += _dot(oh, jnp.ones((nb, 128), jnp.float32))

    @pl.when(i == pl.num_programs(0) - 1)
    def _():
        o_ref[...] = sums_ref[...] / jnp.maximum(cnt_ref[:, :1], 1.0)


def _pool(h, batch):
    nb = 2000
    nblk = N // nb
    b3 = batch.reshape(nblk, 1, nb)
    out = pl.pallas_call(
        _pool_body,
        grid=(nblk,),
        in_specs=[
            pl.BlockSpec((nb, HP), lambda i: (i, 0)),
            pl.BlockSpec((1, 1, nb), lambda i: (i, 0, 0)),
        ],
        out_specs=pl.BlockSpec((NG, HP), lambda i: (0, 0)),
        out_shape=jax.ShapeDtypeStruct((NG, HP), jnp.float32),
        scratch_shapes=[
            pltpu.VMEM((NG, HP), jnp.float32),
            pltpu.VMEM((NG, 128), jnp.float32),
        ],
    )(h, b3)
    return out[:, :H]


@functools.lru_cache(maxsize=None)
def _make_edge_sc(write_r, k):
    mesh = plsc.VectorSubcoreMesh(core_axis_name="c", subcore_axis_name="s")
    out_type = [jax.ShapeDtypeStruct((2, NACC, 2 * QW), jnp.float32)]
    if write_r:
        out_type = [jax.ShapeDtypeStruct((2, E, QW), jnp.float32)] + out_type
    slot_scratch = [
        pltpu.VMEM((2, B), jnp.int32),
        pltpu.VMEM((B,), jnp.int32),
        pltpu.VMEM((B,), jnp.int32),
        pltpu.VMEM((B,), jnp.int32),
        pltpu.VMEM((B, QW), jnp.float32),
        pltpu.VMEM((B, 2 * QW), jnp.float32),
        pltpu.VMEM((B, QW), jnp.float32),
        pltpu.VMEM((B, QW), jnp.float32),
        pltpu.VMEM((B, 2 * QW), jnp.float32),
        pltpu.SemaphoreType.DMA,
        pltpu.SemaphoreType.DMA,
    ]
    scratch_types = slot_scratch + slot_scratch + [
        pltpu.VMEM_SHARED((NACC, 2 * QW), jnp.float32),
    ]
    nslot = len(slot_scratch)

    @functools.partial(pl.kernel, mesh=mesh, out_type=out_type,
                       scratch_types=scratch_types,
                       compiler_params=pltpu.CompilerParams(
                           use_tc_tiling_on_sc=False))
    def kern(idx_hbm, at_hbm, bvt_hbm, ch_hbm, *refs):
        if write_r:
            r_hbm, acc_hbm = refs[0], refs[1]
            refs = refs[2:]
        else:
            acc_hbm = refs[0]
            refs = refs[1:]
        slots = [refs[:nslot], refs[nslot:2 * nslot]]
        accum = refs[2 * nslot]
        c = lax.axis_index("c")
        s = lax.axis_index("s")
        cn = (2 * k + c) * N
        base0 = s * EPT

        zsrc = slots[0][8]

        def zb(i, carry):
            for j in range(2 * QW // 16):
                zsrc[i, pl.ds(j * 16, 16)] = jnp.zeros((16,), jnp.float32)
            return carry

        lax.fori_loop(0, B, zb, 0)
        for j in range(TPR // B):
            pltpu.sync_copy(zsrc, accum.at[pl.ds(s * TPR + j * B, B)])
        plsc.subcore_barrier()

        def issue(slot, b):
            idxb, aidx, bidx, didx = slot[0], slot[1], slot[2], slot[3]
            arows, bvrows, chb = slot[4], slot[5], slot[6]
            gsem = slot[9]
            base = base0 + b * B
            pltpu.sync_copy(idx_hbm.at[:, pl.ds(base, B)], idxb)
            for j in range(B // 16):
                dsj = pl.ds(j * 16, 16)
                d = idxb[0, dsj]
                didx[dsj] = d
                aidx[dsj] = d + cn
                bidx[dsj] = idxb[1, dsj] + cn
            pltpu.async_copy(at_hbm.at[aidx], arows, gsem)
            pltpu.async_copy(bvt_hbm.at[bidx], bvrows, gsem)
            pltpu.async_copy(ch_hbm.at[c, pl.ds(base, B)], chb, gsem)

        def wait_gathers(slot):
            pltpu.make_async_copy(at_hbm.at[slot[1]], slot[4], slot[9]).wait()
            pltpu.make_async_copy(bvt_hbm.at[slot[2]], slot[5], slot[9]).wait()
            pltpu.make_async_copy(ch_hbm.at[c, pl.ds(0, B)], slot[6], slot[9]).wait()

        def compute(slot, b):
            arows, bvrows, chb = slot[4], slot[5], slot[6]
            rbuf, contrib, didx, wsem = slot[7], slot[8], slot[3], slot[10]

            @plsc.parallel_loop(0, B, 1, unroll=8)
            def edge(i):
                for kk in range(QW // 16):
                    dk = pl.ds(kk * 16, 16)
                    dk2 = pl.ds(QW + kk * 16, 16)
                    eh = arows[i, dk] + bvrows[i, dk] + chb[i, dk]
                    sg = 1.0 / (1.0 + jnp.exp(-eh))
                    if write_r:
                        rbuf[i, dk] = jnp.maximum(eh, 0.0)
                    contrib[i, dk] = sg * bvrows[i, dk2]
                    contrib[i, dk2] = sg

            base = base0 + b * B
            if write_r:
                pltpu.sync_copy(rbuf, r_hbm.at[c, pl.ds(base, B)])
            pltpu.sync_copy(contrib, accum.at[didx], add=True)

        nb2 = NBLK // 2
        issue(slots[0], 0)

        def outer(g, carry):
            b0 = 2 * g
            issue(slots[1], b0 + 1)
            wait_gathers(slots[0])
            compute(slots[0], b0)
            issue(slots[0], lax.min(b0 + 2, NBLK - 2))
            wait_gathers(slots[1])
            compute(slots[1], b0 + 1)
            return carry

        lax.fori_loop(0, nb2, outer, 0)
        wait_gathers(slots[0])
        plsc.subcore_barrier()
        pltpu.sync_copy(accum.at[pl.ds(s * TPR, TPR)],
                        acc_hbm.at[c, pl.ds(s * TPR, TPR)])

    return kern


def _edge_sc(dst, src, at4, bvt4, ch, write_r, k):
    idx2 = jnp.stack([dst, src])
    res = _make_edge_sc(write_r, k)(idx2, at4, bvt4, ch[2 * k:2 * k + 2])
    if write_r:
        return res[0], res[1]
    res = res[0] if isinstance(res, (list, tuple)) else res
    return None, res


def kernel(x, edge_index, batch, lap_enc, edge_attr, Wh, bh, Wp, bp, We, be,
           A, bA, Bm, bB, C, bC, U, bU, V, bV, g1h, b1h, g1e, b1e,
           W1, bf1, W2, bf2, g2, b2):
    src = edge_index[0]
    dst = edge_index[1]

    whp = _pad2(Wh, Wh.shape[0], HP)
    wpp = _pad2(Wp, Wp.shape[0], HP)
    b0 = _padb(bh + bp, HP)
    wep = _pad2(We, 40, HP)
    bep = _padb(be, HP)
    eap = jnp.pad(edge_attr, ((0, 0), (0, 40 - edge_attr.shape[1])))

    h = _embed(x, lap_enc, whp, wpp, b0)

    eprev = None
    ra = rb = None
    for l in range(3):
        ap = _pad2(A[l], HP, HP)
        bmp = _pad2(Bm[l], HP, HP)
        cp = _pad2(C[l], HP, HP)
        up = _pad2(U[l], HP, HP)
        vp = _pad2(V[l], HP, HP)
        at, bvt, uh = _tables(h, ap, _padb(bA[l], HP), bmp, _padb(bB[l], HP),
                              vp, _padb(bV[l], HP), up, _padb(bU[l], HP))
        if l == 0:
            eprev, ch = _e0(eap, wep, bep, cp, _padb(bC[l], HP))
        else:
            eprev, ch = _eres(eprev, ra, rb, _padb(g1e[l], HP), _padb(b1e[l], HP),
                              cp, _padb(bC[l], HP), write_e=(l < 2))
        at4 = at.reshape(4 * N, QW)
        bvt4 = bvt.reshape(4 * N, 2 * QW)
        write_r = l < 2
        ra, aa = _edge_sc(dst, src, at4, bvt4, ch, write_r, 0)
        rb, ab = _edge_sc(dst, src, at4, bvt4, ch, write_r, 1)
        h = _hupd(h, uh, aa, ab, _padb(g1h[l], HP), _padb(b1h[l], HP),
                  _pad2(W1[l], HP, FP), _padb(bf1[l], FP),
                  _pad2(W2[l], FP, HP), _padb(bf2[l], HP),
                  _padb(g2[l], HP), _padb(b2[l], HP))

    return _pool(h, batch)


# speedup vs baseline: 3.1076x; 1.4198x over previous
import functools

import jax
import jax.numpy as jnp
from jax import lax
from jax.experimental import pallas as pl
from jax.experimental.pallas import tpu as pltpu
from jax.experimental.pallas import tpu_sc as plsc

N = 10000
E = 320000
H = 166
HP = 176
FP = 336
NG = 32
QW = 48
Q3 = H - 3 * QW
NT = 16
EPT = E // NT
B = 80
NBLK = EPT // B
NACC = 10240
TPR = NACC // NT


def _pad2(w, r, c):
    return jnp.pad(w, ((0, r - w.shape[0]), (0, c - w.shape[1])))


def _padb(b, c):
    return jnp.pad(b, (0, c - b.shape[0])).reshape(1, c)


def _split48(p):
    n = p.shape[0]
    qs = [p[:, 0:48], p[:, 48:96], p[:, 96:144],
          jnp.concatenate([p[:, 144:176], jnp.zeros((n, 16), jnp.float32)], axis=1)]
    return jnp.stack(qs, axis=0)


def _col_mask():
    return (lax.broadcasted_iota(jnp.int32, (1, HP), 1) < H).astype(jnp.float32)


def _ln_masked(v, g, b):
    m = jnp.sum(v, axis=1, keepdims=True) * (1.0 / H)
    sq = jnp.sum(v * v, axis=1, keepdims=True) * (1.0 / H)
    inv = lax.rsqrt(sq - m * m + 1e-5)
    return ((v - m) * inv * g + b) * _col_mask()


def _dot(a, b):
    return jnp.dot(a, b, preferred_element_type=jnp.float32)


def _embed_body(x_ref, lp_ref, wh_ref, wp_ref, b_ref, out_ref):
    out_ref[...] = (_dot(x_ref[...], wh_ref[...])
                    + _dot(lp_ref[...], wp_ref[...]) + b_ref[...])


def _embed(x, lap, whp, wpp, b0):
    nb = 1000
    return pl.pallas_call(
        _embed_body,
        grid=(N // nb,),
        in_specs=[
            pl.BlockSpec((nb, x.shape[1]), lambda i: (i, 0)),
            pl.BlockSpec((nb, lap.shape[1]), lambda i: (i, 0)),
            pl.BlockSpec(whp.shape, lambda i: (0, 0)),
            pl.BlockSpec(wpp.shape, lambda i: (0, 0)),
            pl.BlockSpec((1, HP), lambda i: (0, 0)),
        ],
        out_specs=pl.BlockSpec((nb, HP), lambda i: (i, 0)),
        out_shape=jax.ShapeDtypeStruct((N, HP), jnp.float32),
    )(x, lap, whp, wpp, b0)


def _tables_body(h_ref, a_ref, ba_ref, bm_ref, bb_ref, v_ref, bv_ref, u_ref, bu_ref,
                 at_ref, bvt_ref, uh_ref):
    hb = h_ref[...]
    pa = _dot(hb, a_ref[...]) + ba_ref[...]
    pb = _dot(hb, bm_ref[...]) + bb_ref[...]
    pv = _dot(hb, v_ref[...]) + bv_ref[...]
    pu = _dot(hb, u_ref[...]) + bu_ref[...]
    at_ref[...] = _split48(pa)
    sb = _split48(pb)
    sv = _split48(pv)
    bvt_ref[...] = jnp.concatenate([sb, sv], axis=2)
    uh_ref[...] = pu


def _tables(h, ap, bap, bmp, bbp, vp, bvp, up, bup):
    nb = 1000
    w = pl.BlockSpec((HP, HP), lambda i: (0, 0))
    bspec = pl.BlockSpec((1, HP), lambda i: (0, 0))
    return pl.pallas_call(
        _tables_body,
        grid=(N // nb,),
        in_specs=[pl.BlockSpec((nb, HP), lambda i: (i, 0)),
                  w, bspec, w, bspec, w, bspec, w, bspec],
        out_specs=[
            pl.BlockSpec((4, nb, QW), lambda i: (0, i, 0)),
            pl.BlockSpec((4, nb, 2 * QW), lambda i: (0, i, 0)),
            pl.BlockSpec((nb, HP), lambda i: (i, 0)),
        ],
        out_shape=[
            jax.ShapeDtypeStruct((4, N, QW), jnp.float32),
            jax.ShapeDtypeStruct((4, N, 2 * QW), jnp.float32),
            jax.ShapeDtypeStruct((N, HP), jnp.float32),
        ],
    )(h, ap, bap, bmp, bbp, vp, bvp, up, bup)


def _e0_body(ea_ref, we_ref, be_ref, c_ref, bc_ref, e0_ref, ch_ref):
    eb = _dot(ea_ref[...], we_ref[...]) + be_ref[...]
    e0_ref[...] = eb
    ch_ref[...] = _dot(eb, c_ref[...]) + bc_ref[...]


def _e0(eap, wep, bep, cp, bcp):
    eb = 2000
    return pl.pallas_call(
        _e0_body,
        grid=(E // eb,),
        in_specs=[
            pl.BlockSpec((eb, eap.shape[1]), lambda i: (i, 0)),
            pl.BlockSpec(wep.shape, lambda i: (0, 0)),
            pl.BlockSpec((1, HP), lambda i: (0, 0)),
            pl.BlockSpec((HP, 4 * QW), lambda i: (0, 0)),
            pl.BlockSpec((1, 4 * QW), lambda i: (0, 0)),
        ],
        out_specs=[
            pl.BlockSpec((eb, HP), lambda i: (i, 0)),
            pl.BlockSpec((eb, 4 * QW), lambda i: (i, 0)),
        ],
        out_shape=[
            jax.ShapeDtypeStruct((E, HP), jnp.float32),
            jax.ShapeDtypeStruct((E, 4 * QW), jnp.float32),
        ],
    )(eap, wep, bep, cp, bcp)


def _make_eres_body(write_e):
    def body(ep_ref, ra_ref, rb_ref, g_ref, b_ref, c_ref, bc_ref, *outs):
        if write_e:
            e_ref, ch_ref = outs
        else:
            (ch_ref,) = outs
        rp = jnp.concatenate([ra_ref[...], rb_ref[:, :HP - 2 * QW]], axis=1)
        v = ep_ref[...] + rp
        en = _ln_masked(v, g_ref[...], b_ref[...])
        if write_e:
            e_ref[...] = en
        ch_ref[...] = _dot(en, c_ref[...]) + bc_ref[...]
    return body


def _eres(eprev, ra, rb, gp, bp_, cp, bcp, write_e):
    eb = 2000
    out_specs = [pl.BlockSpec((eb, 4 * QW), lambda i: (i, 0))]
    out_shape = [jax.ShapeDtypeStruct((E, 4 * QW), jnp.float32)]
    if write_e:
        out_specs = [pl.BlockSpec((eb, HP), lambda i: (i, 0))] + out_specs
        out_shape = [jax.ShapeDtypeStruct((E, HP), jnp.float32)] + out_shape
    res = pl.pallas_call(
        _make_eres_body(write_e),
        grid=(E // eb,),
        in_specs=[
            pl.BlockSpec((eb, HP), lambda i: (i, 0)),
            pl.BlockSpec((eb, 2 * QW), lambda i: (i, 0)),
            pl.BlockSpec((eb, 2 * QW), lambda i: (i, 0)),
            pl.BlockSpec((1, HP), lambda i: (0, 0)),
            pl.BlockSpec((1, HP), lambda i: (0, 0)),
            pl.BlockSpec((HP, 4 * QW), lambda i: (0, 0)),
            pl.BlockSpec((1, 4 * QW), lambda i: (0, 0)),
        ],
        out_specs=out_specs,
        out_shape=out_shape,
    )(eprev, ra, rb, gp, bp_, cp, bcp)
    if write_e:
        return res[0], res[1]
    return None, res[0]


def _hupd_body(h_ref, uh_ref, aa_ref, ab_ref, g1_ref, b1_ref, w1_ref, bf1_ref,
               w2_ref, bf2_ref, g2_ref, b2_ref, out_ref):
    n = aa_ref.shape[0]
    z10 = jnp.zeros((n, HP - H), jnp.float32)
    num = jnp.concatenate(
        [aa_ref[:, :2 * QW], ab_ref[:, :H - 2 * QW], z10], axis=1)
    den = jnp.concatenate(
        [aa_ref[:, 2 * QW:4 * QW], ab_ref[:, 2 * QW:2 * QW + H - 2 * QW], z10],
        axis=1)
    hn = jnp.maximum(uh_ref[...] + num / (den + 1e-6), 0.0)
    h1 = _ln_masked(h_ref[...] + hn, g1_ref[...], b1_ref[...])
    h2 = jnp.maximum(_dot(h1, w1_ref[...]) + bf1_ref[...], 0.0)
    h2 = _dot(h2, w2_ref[...]) + bf2_ref[...]
    out_ref[...] = _ln_masked(h1 + h2, g2_ref[...], b2_ref[...])


def _hupd(h, uh, aa, ab, g1p, b1p, w1p, bf1p, w2p, bf2p, g2p, b2p):
    nb = 1000
    bspec = pl.BlockSpec((1, HP), lambda i: (0, 0))
    return pl.pallas_call(
        _hupd_body,
        grid=(N // nb,),
        in_specs=[
            pl.BlockSpec((nb, HP), lambda i: (i, 0)),
            pl.BlockSpec((nb, HP), lambda i: (i, 0)),
            pl.BlockSpec((nb, 4 * QW), lambda i: (i, 0)),
            pl.BlockSpec((nb, 4 * QW), lambda i: (i, 0)),
            bspec, bspec,
            pl.BlockSpec((HP, FP), lambda i: (0, 0)),
            pl.BlockSpec((1, FP), lambda i: (0, 0)),
            pl.BlockSpec((FP, HP), lambda i: (0, 0)),
            bspec, bspec, bspec,
        ],
        out_specs=pl.BlockSpec((nb, HP), lambda i: (i, 0)),
        out_shape=jax.ShapeDtypeStruct((N, HP), jnp.float32),
    )(h, uh, aa, ab, g1p, b1p, w1p, bf1p, w2p, bf2p, g2p, b2p)


def _pool_body(h_ref, b_ref, o_ref, sums_ref, cnt_ref):
    i = pl.program_id(0)

    @pl.when(i == 0)
    def _():
        sums_ref[...] = jnp.zeros_like(sums_ref)
        cnt_ref[...] = jnp.zeros_like(cnt_ref)

    bi = b_ref[0, 0, :]
    nb = bi.shape[0]
    oh = (bi[None, :] == lax.broadcasted_iota(jnp.int32, (NG, nb), 0)).astype(jnp.float32)
    sums_ref[...] += _dot(oh, h_ref[...])
    cnt_ref[...] += _dot(oh, jnp.ones((nb, 128), jnp.float32))

    @pl.when(i == pl.num_programs(0) - 1)
    def _():
        o_ref[...] = sums_ref[...] / jnp.maximum(cnt_ref[:, :1], 1.0)


def _pool(h, batch):
    nb = 2000
    nblk = N // nb
    b3 = batch.reshape(nblk, 1, nb)
    out = pl.pallas_call(
        _pool_body,
        grid=(nblk,),
        in_specs=[
            pl.BlockSpec((nb, HP), lambda i: (i, 0)),
            pl.BlockSpec((1, 1, nb), lambda i: (i, 0, 0)),
        ],
        out_specs=pl.BlockSpec((NG, HP), lambda i: (0, 0)),
        out_shape=jax.ShapeDtypeStruct((NG, HP), jnp.float32),
        scratch_shapes=[
            pltpu.VMEM((NG, HP), jnp.float32),
            pltpu.VMEM((NG, 128), jnp.float32),
        ],
    )(h, b3)
    return out[:, :H]


@functools.lru_cache(maxsize=None)
def _make_edge_sc(write_r, k):
    mesh = plsc.VectorSubcoreMesh(core_axis_name="c", subcore_axis_name="s")
    out_type = [jax.ShapeDtypeStruct((NACC, 4 * QW), jnp.float32)]
    if write_r:
        out_type = [jax.ShapeDtypeStruct((E, 2 * QW), jnp.float32)] + out_type
    slot_scratch = [
        pltpu.VMEM((2, B), jnp.int32),
        pltpu.VMEM((B,), jnp.int32),
        pltpu.VMEM((B,), jnp.int32),
        pltpu.VMEM((B,), jnp.int32),
        pltpu.VMEM((B, QW), jnp.float32),
        pltpu.VMEM((B, 2 * QW), jnp.float32),
        pltpu.VMEM((B, QW), jnp.float32),
        pltpu.VMEM((B, QW), jnp.float32),
        pltpu.VMEM((B, 2 * QW), jnp.float32),
        pltpu.SemaphoreType.DMA,
        pltpu.SemaphoreType.DMA,
    ]
    scratch_types = slot_scratch + slot_scratch + [
        pltpu.VMEM_SHARED((NACC, 2 * QW), jnp.float32),
    ]
    nslot = len(slot_scratch)

    @functools.partial(pl.kernel, mesh=mesh, out_type=out_type,
                       scratch_types=scratch_types,
                       compiler_params=pltpu.CompilerParams(
                           use_tc_tiling_on_sc=False))
    def kern(idx_hbm, at_hbm, bvt_hbm, ch_hbm, *refs):
        if write_r:
            r_hbm, acc_hbm = refs[0], refs[1]
            refs = refs[2:]
        else:
            acc_hbm = refs[0]
            refs = refs[1:]
        slots = [refs[:nslot], refs[nslot:2 * nslot]]
        accum = refs[2 * nslot]
        c = lax.axis_index("c")
        s = lax.axis_index("s")
        cn = (2 * k + c) * N
        qoff = (2 * k + c) * QW
        roff = c * QW
        base0 = s * EPT

        zsrc = slots[0][8]

        def zb(i, carry):
            for j in range(2 * QW // 16):
                zsrc[i, pl.ds(j * 16, 16)] = jnp.zeros((16,), jnp.float32)
            return carry

        lax.fori_loop(0, B, zb, 0)
        for j in range(TPR // B):
            pltpu.sync_copy(zsrc, accum.at[pl.ds(s * TPR + j * B, B)])
        plsc.subcore_barrier()

        def issue(slot, b):
            idxb, aidx, bidx, didx = slot[0], slot[1], slot[2], slot[3]
            arows, bvrows, chb = slot[4], slot[5], slot[6]
            gsem = slot[9]
            base = base0 + b * B
            pltpu.sync_copy(idx_hbm.at[:, pl.ds(base, B)], idxb)
            for j in range(B // 16):
                dsj = pl.ds(j * 16, 16)
                d = idxb[0, dsj]
                didx[dsj] = d
                aidx[dsj] = d + cn
                bidx[dsj] = idxb[1, dsj] + cn
            pltpu.async_copy(at_hbm.at[aidx], arows, gsem)
            pltpu.async_copy(bvt_hbm.at[bidx], bvrows, gsem)
            pltpu.async_copy(ch_hbm.at[pl.ds(base, B), pl.ds(qoff, QW)], chb, gsem)

        def wait_gathers(slot):
            pltpu.make_async_copy(at_hbm.at[slot[1]], slot[4], slot[9]).wait()
            pltpu.make_async_copy(bvt_hbm.at[slot[2]], slot[5], slot[9]).wait()
            pltpu.make_async_copy(ch_hbm.at[pl.ds(0, B), pl.ds(0, QW)],
                                  slot[6], slot[9]).wait()

        def compute(slot, b):
            arows, bvrows, chb = slot[4], slot[5], slot[6]
            rbuf, contrib, didx, wsem = slot[7], slot[8], slot[3], slot[10]

            @plsc.parallel_loop(0, B, 1, unroll=8)
            def edge(i):
                for kk in range(QW // 16):
                    dk = pl.ds(kk * 16, 16)
                    dk2 = pl.ds(QW + kk * 16, 16)
                    eh = arows[i, dk] + bvrows[i, dk] + chb[i, dk]
                    sg = 1.0 / (1.0 + jnp.exp(-eh))
                    if write_r:
                        rbuf[i, dk] = jnp.maximum(eh, 0.0)
                    contrib[i, dk] = sg * bvrows[i, dk2]
                    contrib[i, dk2] = sg

            base = base0 + b * B
            if write_r:
                pltpu.sync_copy(rbuf, r_hbm.at[pl.ds(base, B), pl.ds(roff, QW)])
            pltpu.sync_copy(contrib, accum.at[didx], add=True)

        nb2 = NBLK // 2
        issue(slots[0], 0)

        def outer(g, carry):
            b0 = 2 * g
            issue(slots[1], b0 + 1)
            wait_gathers(slots[0])
            compute(slots[0], b0)
            issue(slots[0], lax.min(b0 + 2, NBLK - 2))
            wait_gathers(slots[1])
            compute(slots[1], b0 + 1)
            return carry

        lax.fori_loop(0, nb2, outer, 0)
        wait_gathers(slots[0])
        plsc.subcore_barrier()
        rows = pl.ds(s * TPR, TPR)
        pltpu.sync_copy(accum.at[rows, pl.ds(0, QW)],
                        acc_hbm.at[rows, pl.ds(roff, QW)])
        pltpu.sync_copy(accum.at[rows, pl.ds(QW, QW)],
                        acc_hbm.at[rows, pl.ds(2 * QW + roff, QW)])

    return kern


def _edge_sc(dst, src, at4, bvt4, ch, write_r, k):
    idx2 = jnp.stack([dst, src])
    res = _make_edge_sc(write_r, k)(idx2, at4, bvt4, ch)
    if write_r:
        return res[0], res[1]
    res = res[0] if isinstance(res, (list, tuple)) else res
    return None, res


def kernel(x, edge_index, batch, lap_enc, edge_attr, Wh, bh, Wp, bp, We, be,
           A, bA, Bm, bB, C, bC, U, bU, V, bV, g1h, b1h, g1e, b1e,
           W1, bf1, W2, bf2, g2, b2):
    src = edge_index[0]
    dst = edge_index[1]

    whp = _pad2(Wh, Wh.shape[0], HP)
    wpp = _pad2(Wp, Wp.shape[0], HP)
    b0 = _padb(bh + bp, HP)
    wep = _pad2(We, 40, HP)
    bep = _padb(be, HP)
    eap = jnp.pad(edge_attr, ((0, 0), (0, 40 - edge_attr.shape[1])))

    h = _embed(x, lap_enc, whp, wpp, b0)

    eprev = None
    ra = rb = None
    for l in range(3):
        ap = _pad2(A[l], HP, HP)
        bmp = _pad2(Bm[l], HP, HP)
        cp = _pad2(C[l], HP, 4 * QW)
        up = _pad2(U[l], HP, HP)
        vp = _pad2(V[l], HP, HP)
        at, bvt, uh = _tables(h, ap, _padb(bA[l], HP), bmp, _padb(bB[l], HP),
                              vp, _padb(bV[l], HP), up, _padb(bU[l], HP))
        if l == 0:
            eprev, ch = _e0(eap, wep, bep, cp, _padb(bC[l], 4 * QW))
        else:
            eprev, ch = _eres(eprev, ra, rb, _padb(g1e[l], HP), _padb(b1e[l], HP),
                              cp, _padb(bC[l], 4 * QW), write_e=(l < 2))
        at4 = at.reshape(4 * N, QW)
        bvt4 = bvt.reshape(4 * N, 2 * QW)
        write_r = l < 2
        ra, aa = _edge_sc(dst, src, at4, bvt4, ch, write_r, 0)
        rb, ab = _edge_sc(dst, src, at4, bvt4, ch, write_r, 1)
        h = _hupd(h, uh, aa, ab, _padb(g1h[l], HP), _padb(b1h[l], HP),
                  _pad2(W1[l], HP, FP), _padb(bf1[l], FP),
                  _pad2(W2[l], FP, HP), _padb(bf2[l], HP),
                  _padb(g2[l], HP), _padb(b2[l], HP))

    return _pool(h, batch)


# speedup vs baseline: 3.1693x; 1.0199x over previous
import functools

import jax
import jax.numpy as jnp
from jax import lax
from jax.experimental import pallas as pl
from jax.experimental.pallas import tpu as pltpu
from jax.experimental.pallas import tpu_sc as plsc

N = 10000
E = 320000
H = 166
HP = 176
FP = 336
NG = 32
QW = 48
Q3 = H - 3 * QW
NT = 16
EPT = E // NT
B = 80
NBLK = EPT // B
NACC = 10240
TPR = NACC // NT


def _pad2(w, r, c):
    return jnp.pad(w, ((0, r - w.shape[0]), (0, c - w.shape[1])))


def _padb(b, c):
    return jnp.pad(b, (0, c - b.shape[0])).reshape(1, c)


def _split48(p):
    n = p.shape[0]
    qs = [p[:, 0:48], p[:, 48:96], p[:, 96:144],
          jnp.concatenate([p[:, 144:176], jnp.zeros((n, 16), jnp.float32)], axis=1)]
    return jnp.stack(qs, axis=0)


def _col_mask():
    return (lax.broadcasted_iota(jnp.int32, (1, HP), 1) < H).astype(jnp.float32)


def _ln_masked(v, g, b):
    m = jnp.sum(v, axis=1, keepdims=True) * (1.0 / H)
    sq = jnp.sum(v * v, axis=1, keepdims=True) * (1.0 / H)
    inv = lax.rsqrt(sq - m * m + 1e-5)
    return ((v - m) * inv * g + b) * _col_mask()


def _dot(a, b):
    return jnp.dot(a, b, preferred_element_type=jnp.float32)


def _embed_body(x_ref, lp_ref, wh_ref, wp_ref, b_ref, out_ref):
    out_ref[...] = (_dot(x_ref[...], wh_ref[...])
                    + _dot(lp_ref[...], wp_ref[...]) + b_ref[...])


def _embed(x, lap, whp, wpp, b0):
    nb = 1000
    return pl.pallas_call(
        _embed_body,
        grid=(N // nb,),
        in_specs=[
            pl.BlockSpec((nb, x.shape[1]), lambda i: (i, 0)),
            pl.BlockSpec((nb, lap.shape[1]), lambda i: (i, 0)),
            pl.BlockSpec(whp.shape, lambda i: (0, 0)),
            pl.BlockSpec(wpp.shape, lambda i: (0, 0)),
            pl.BlockSpec((1, HP), lambda i: (0, 0)),
        ],
        out_specs=pl.BlockSpec((nb, HP), lambda i: (i, 0)),
        out_shape=jax.ShapeDtypeStruct((N, HP), jnp.float32),
    )(x, lap, whp, wpp, b0)


def _tables_body(h_ref, a_ref, ba_ref, bm_ref, bb_ref, v_ref, bv_ref, u_ref, bu_ref,
                 at_ref, bvt_ref, uh_ref):
    hb = h_ref[...]
    pa = _dot(hb, a_ref[...]) + ba_ref[...]
    pb = _dot(hb, bm_ref[...]) + bb_ref[...]
    pv = _dot(hb, v_ref[...]) + bv_ref[...]
    pu = _dot(hb, u_ref[...]) + bu_ref[...]
    at_ref[...] = _split48(pa)
    sb = _split48(pb)
    sv = _split48(pv)
    bvt_ref[...] = jnp.concatenate([sb, sv], axis=2)
    uh_ref[...] = pu


def _tables(h, ap, bap, bmp, bbp, vp, bvp, up, bup):
    nb = 1000
    w = pl.BlockSpec((HP, HP), lambda i: (0, 0))
    bspec = pl.BlockSpec((1, HP), lambda i: (0, 0))
    return pl.pallas_call(
        _tables_body,
        grid=(N // nb,),
        in_specs=[pl.BlockSpec((nb, HP), lambda i: (i, 0)),
                  w, bspec, w, bspec, w, bspec, w, bspec],
        out_specs=[
            pl.BlockSpec((4, nb, QW), lambda i: (0, i, 0)),
            pl.BlockSpec((4, nb, 2 * QW), lambda i: (0, i, 0)),
            pl.BlockSpec((nb, HP), lambda i: (i, 0)),
        ],
        out_shape=[
            jax.ShapeDtypeStruct((4, N, QW), jnp.float32),
            jax.ShapeDtypeStruct((4, N, 2 * QW), jnp.float32),
            jax.ShapeDtypeStruct((N, HP), jnp.float32),
        ],
    )(h, ap, bap, bmp, bbp, vp, bvp, up, bup)


def _e0_body(ea_ref, we_ref, be_ref, c_ref, bc_ref, e0_ref, ch_ref):
    eb = _dot(ea_ref[...], we_ref[...]) + be_ref[...]
    e0_ref[...] = eb
    ch_ref[...] = _dot(eb, c_ref[...]) + bc_ref[...]


def _e0(eap, wep, bep, cp, bcp):
    eb = 2000
    return pl.pallas_call(
        _e0_body,
        grid=(E // eb,),
        in_specs=[
            pl.BlockSpec((eb, eap.shape[1]), lambda i: (i, 0)),
            pl.BlockSpec(wep.shape, lambda i: (0, 0)),
            pl.BlockSpec((1, HP), lambda i: (0, 0)),
            pl.BlockSpec((HP, 4 * QW), lambda i: (0, 0)),
            pl.BlockSpec((1, 4 * QW), lambda i: (0, 0)),
        ],
        out_specs=[
            pl.BlockSpec((eb, HP), lambda i: (i, 0)),
            pl.BlockSpec((eb, 4 * QW), lambda i: (i, 0)),
        ],
        out_shape=[
            jax.ShapeDtypeStruct((E, HP), jnp.float32),
            jax.ShapeDtypeStruct((E, 4 * QW), jnp.float32),
        ],
    )(eap, wep, bep, cp, bcp)


def _make_eres_body(write_e):
    def body(ep_ref, ra_ref, rb_ref, g_ref, b_ref, c_ref, bc_ref, *outs):
        if write_e:
            e_ref, ch_ref = outs
        else:
            (ch_ref,) = outs
        rp = jnp.concatenate([ra_ref[...], rb_ref[:, :HP - 2 * QW]], axis=1)
        v = ep_ref[...] + rp
        en = _ln_masked(v, g_ref[...], b_ref[...])
        if write_e:
            e_ref[...] = en
        ch_ref[...] = _dot(en, c_ref[...]) + bc_ref[...]
    return body


def _eres(eprev, ra, rb, gp, bp_, cp, bcp, write_e):
    eb = 2000
    out_specs = [pl.BlockSpec((eb, 4 * QW), lambda i: (i, 0))]
    out_shape = [jax.ShapeDtypeStruct((E, 4 * QW), jnp.float32)]
    if write_e:
        out_specs = [pl.BlockSpec((eb, HP), lambda i: (i, 0))] + out_specs
        out_shape = [jax.ShapeDtypeStruct((E, HP), jnp.float32)] + out_shape
    res = pl.pallas_call(
        _make_eres_body(write_e),
        grid=(E // eb,),
        in_specs=[
            pl.BlockSpec((eb, HP), lambda i: (i, 0)),
            pl.BlockSpec((eb, 2 * QW), lambda i: (i, 0)),
            pl.BlockSpec((eb, 2 * QW), lambda i: (i, 0)),
            pl.BlockSpec((1, HP), lambda i: (0, 0)),
            pl.BlockSpec((1, HP), lambda i: (0, 0)),
            pl.BlockSpec((HP, 4 * QW), lambda i: (0, 0)),
            pl.BlockSpec((1, 4 * QW), lambda i: (0, 0)),
        ],
        out_specs=out_specs,
        out_shape=out_shape,
    )(eprev, ra, rb, gp, bp_, cp, bcp)
    if write_e:
        return res[0], res[1]
    return None, res[0]


def _hupd_body(h_ref, uh_ref, aa_ref, ab_ref, g1_ref, b1_ref, w1_ref, bf1_ref,
               w2_ref, bf2_ref, g2_ref, b2_ref, out_ref):
    n = aa_ref.shape[0]
    z10 = jnp.zeros((n, HP - H), jnp.float32)
    num = jnp.concatenate(
        [aa_ref[:, :2 * QW], ab_ref[:, :H - 2 * QW], z10], axis=1)
    den = jnp.concatenate(
        [aa_ref[:, 2 * QW:4 * QW], ab_ref[:, 2 * QW:2 * QW + H - 2 * QW], z10],
        axis=1)
    hn = jnp.maximum(uh_ref[...] + num / (den + 1e-6), 0.0)
    h1 = _ln_masked(h_ref[...] + hn, g1_ref[...], b1_ref[...])
    h2 = jnp.maximum(_dot(h1, w1_ref[...]) + bf1_ref[...], 0.0)
    h2 = _dot(h2, w2_ref[...]) + bf2_ref[...]
    out_ref[...] = _ln_masked(h1 + h2, g2_ref[...], b2_ref[...])


def _hupd(h, uh, aa, ab, g1p, b1p, w1p, bf1p, w2p, bf2p, g2p, b2p):
    nb = 1000
    bspec = pl.BlockSpec((1, HP), lambda i: (0, 0))
    return pl.pallas_call(
        _hupd_body,
        grid=(N // nb,),
        in_specs=[
            pl.BlockSpec((nb, HP), lambda i: (i, 0)),
            pl.BlockSpec((nb, HP), lambda i: (i, 0)),
            pl.BlockSpec((nb, 4 * QW), lambda i: (i, 0)),
            pl.BlockSpec((nb, 4 * QW), lambda i: (i, 0)),
            bspec, bspec,
            pl.BlockSpec((HP, FP), lambda i: (0, 0)),
            pl.BlockSpec((1, FP), lambda i: (0, 0)),
            pl.BlockSpec((FP, HP), lambda i: (0, 0)),
            bspec, bspec, bspec,
        ],
        out_specs=pl.BlockSpec((nb, HP), lambda i: (i, 0)),
        out_shape=jax.ShapeDtypeStruct((N, HP), jnp.float32),
    )(h, uh, aa, ab, g1p, b1p, w1p, bf1p, w2p, bf2p, g2p, b2p)


def _pool_body(h_ref, b_ref, o_ref, sums_ref, cnt_ref):
    i = pl.program_id(0)

    @pl.when(i == 0)
    def _():
        sums_ref[...] = jnp.zeros_like(sums_ref)
        cnt_ref[...] = jnp.zeros_like(cnt_ref)

    bi = b_ref[0, 0, :]
    nb = bi.shape[0]
    oh = (bi[None, :] == lax.broadcasted_iota(jnp.int32, (NG, nb), 0)).astype(jnp.float32)
    sums_ref[...] += _dot(oh, h_ref[...])
    cnt_ref[...] += _dot(oh, jnp.ones((nb, 128), jnp.float32))

    @pl.when(i == pl.num_programs(0) - 1)
    def _():
        o_ref[...] = sums_ref[...] / jnp.maximum(cnt_ref[:, :1], 1.0)


def _pool(h, batch):
    nb = 2000
    nblk = N // nb
    b3 = batch.reshape(nblk, 1, nb)
    out = pl.pallas_call(
        _pool_body,
        grid=(nblk,),
        in_specs=[
            pl.BlockSpec((nb, HP), lambda i: (i, 0)),
            pl.BlockSpec((1, 1, nb), lambda i: (i, 0, 0)),
        ],
        out_specs=pl.BlockSpec((NG, HP), lambda i: (0, 0)),
        out_shape=jax.ShapeDtypeStruct((NG, HP), jnp.float32),
        scratch_shapes=[
            pltpu.VMEM((NG, HP), jnp.float32),
            pltpu.VMEM((NG, 128), jnp.float32),
        ],
    )(h, b3)
    return out[:, :H]


@functools.lru_cache(maxsize=None)
def _make_edge_sc(write_r, k):
    mesh = plsc.VectorSubcoreMesh(core_axis_name="c", subcore_axis_name="s")
    out_type = [jax.ShapeDtypeStruct((NACC, 4 * QW), jnp.float32)]
    if write_r:
        out_type = [jax.ShapeDtypeStruct((E, 2 * QW), jnp.float32)] + out_type
    slot_scratch = [
        pltpu.VMEM((2, B), jnp.int32),
        pltpu.VMEM((B,), jnp.int32),
        pltpu.VMEM((B,), jnp.int32),
        pltpu.VMEM((B,), jnp.int32),
        pltpu.VMEM((B, QW), jnp.float32),
        pltpu.VMEM((B, 2 * QW), jnp.float32),
        pltpu.VMEM((B, QW), jnp.float32),
        pltpu.VMEM((B, QW), jnp.float32),
        pltpu.VMEM((B, 2 * QW), jnp.float32),
        pltpu.SemaphoreType.DMA,
        pltpu.SemaphoreType.DMA,
    ]
    scratch_types = slot_scratch + slot_scratch + [
        pltpu.VMEM_SHARED((NACC, 2 * QW), jnp.float32),
    ]
    nslot = len(slot_scratch)

    @functools.partial(pl.kernel, mesh=mesh, out_type=out_type,
                       scratch_types=scratch_types,
                       compiler_params=pltpu.CompilerParams(
                           use_tc_tiling_on_sc=False))
    def kern(idx_hbm, at_hbm, bvt_hbm, ch_hbm, *refs):
        if write_r:
            r_hbm, acc_hbm = refs[0], refs[1]
            refs = refs[2:]
        else:
            acc_hbm = refs[0]
            refs = refs[1:]
        slots = [refs[:nslot], refs[nslot:2 * nslot]]
        accum = refs[2 * nslot]
        c = lax.axis_index("c")
        s = lax.axis_index("s")
        cn = (2 * k + c) * N
        qoff = (2 * k + c) * QW
        roff = c * QW
        base0 = s * EPT

        zsrc = slots[0][8]

        def zb(i, carry):
            for j in range(2 * QW // 16):
                zsrc[i, pl.ds(j * 16, 16)] = jnp.zeros((16,), jnp.float32)
            return carry

        lax.fori_loop(0, B, zb, 0)
        for j in range(TPR // B):
            pltpu.sync_copy(zsrc, accum.at[pl.ds(s * TPR + j * B, B)])
        plsc.subcore_barrier()

        def issue(slot, b):
            idxb, aidx, bidx, didx = slot[0], slot[1], slot[2], slot[3]
            arows, bvrows, chb = slot[4], slot[5], slot[6]
            gsem = slot[9]
            base = base0 + b * B
            pltpu.sync_copy(idx_hbm.at[:, pl.ds(base, B)], idxb)
            for j in range(B // 16):
                dsj = pl.ds(j * 16, 16)
                d = idxb[0, dsj]
                didx[dsj] = d
                aidx[dsj] = d + cn
                bidx[dsj] = idxb[1, dsj] + cn
            pltpu.async_copy(at_hbm.at[aidx], arows, gsem)
            pltpu.async_copy(bvt_hbm.at[bidx], bvrows, gsem)
            pltpu.async_copy(ch_hbm.at[pl.ds(base, B), pl.ds(qoff, QW)], chb, gsem)

        def wait_gathers(slot):
            pltpu.make_async_copy(at_hbm.at[slot[1]], slot[4], slot[9]).wait()
            pltpu.make_async_copy(bvt_hbm.at[slot[2]], slot[5], slot[9]).wait()
            pltpu.make_async_copy(ch_hbm.at[pl.ds(0, B), pl.ds(0, QW)],
                                  slot[6], slot[9]).wait()

        def wait_writes(slot):
            pltpu.make_async_copy(
                slot[7], r_hbm.at[pl.ds(0, B), pl.ds(0, QW)], slot[10]).wait()

        def compute(slot, b):
            arows, bvrows, chb = slot[4], slot[5], slot[6]
            rbuf, contrib, didx, wsem = slot[7], slot[8], slot[3], slot[10]
            if write_r:
                wait_writes(slot)

            @plsc.parallel_loop(0, B, 1, unroll=8)
            def edge(i):
                for kk in range(QW // 16):
                    dk = pl.ds(kk * 16, 16)
                    dk2 = pl.ds(QW + kk * 16, 16)
                    eh = arows[i, dk] + bvrows[i, dk] + chb[i, dk]
                    sg = 1.0 / (1.0 + jnp.exp(-eh))
                    if write_r:
                        rbuf[i, dk] = jnp.maximum(eh, 0.0)
                    contrib[i, dk] = sg * bvrows[i, dk2]
                    contrib[i, dk2] = sg

            base = base0 + b * B
            if write_r:
                pltpu.async_copy(rbuf, r_hbm.at[pl.ds(base, B), pl.ds(roff, QW)],
                                 wsem)
            pltpu.sync_copy(contrib, accum.at[didx], add=True)

        nb2 = NBLK // 2
        if write_r:
            src0 = r_hbm.at[pl.ds(0, B), pl.ds(0, QW)]
            pltpu.async_copy(src0, slots[0][7], slots[0][10])
            pltpu.async_copy(src0, slots[1][7], slots[1][10])
        issue(slots[0], 0)

        def outer(g, carry):
            b0 = 2 * g
            issue(slots[1], b0 + 1)
            wait_gathers(slots[0])
            compute(slots[0], b0)
            issue(slots[0], lax.min(b0 + 2, NBLK - 2))
            wait_gathers(slots[1])
            compute(slots[1], b0 + 1)
            return carry

        lax.fori_loop(0, nb2, outer, 0)
        wait_gathers(slots[0])
        if write_r:
            wait_writes(slots[0])
            wait_writes(slots[1])
        plsc.subcore_barrier()
        rows = pl.ds(s * TPR, TPR)
        pltpu.sync_copy(accum.at[rows, pl.ds(0, QW)],
                        acc_hbm.at[rows, pl.ds(roff, QW)])
        pltpu.sync_copy(accum.at[rows, pl.ds(QW, QW)],
                        acc_hbm.at[rows, pl.ds(2 * QW + roff, QW)])

    return kern


def _edge_sc(dst, src, at4, bvt4, ch, write_r, k):
    idx2 = jnp.stack([dst, src])
    res = _make_edge_sc(write_r, k)(idx2, at4, bvt4, ch)
    if write_r:
        return res[0], res[1]
    res = res[0] if isinstance(res, (list, tuple)) else res
    return None, res


def kernel(x, edge_index, batch, lap_enc, edge_attr, Wh, bh, Wp, bp, We, be,
           A, bA, Bm, bB, C, bC, U, bU, V, bV, g1h, b1h, g1e, b1e,
           W1, bf1, W2, bf2, g2, b2):
    src = edge_index[0]
    dst = edge_index[1]

    whp = _pad2(Wh, Wh.shape[0], HP)
    wpp = _pad2(Wp, Wp.shape[0], HP)
    b0 = _padb(bh + bp, HP)
    wep = _pad2(We, 40, HP)
    bep = _padb(be, HP)
    eap = jnp.pad(edge_attr, ((0, 0), (0, 40 - edge_attr.shape[1])))

    h = _embed(x, lap_enc, whp, wpp, b0)

    eprev = None
    ra = rb = None
    for l in range(3):
        ap = _pad2(A[l], HP, HP)
        bmp = _pad2(Bm[l], HP, HP)
        cp = _pad2(C[l], HP, 4 * QW)
        up = _pad2(U[l], HP, HP)
        vp = _pad2(V[l], HP, HP)
        at, bvt, uh = _tables(h, ap, _padb(bA[l], HP), bmp, _padb(bB[l], HP),
                              vp, _padb(bV[l], HP), up, _padb(bU[l], HP))
        if l == 0:
            eprev, ch = _e0(eap, wep, bep, cp, _padb(bC[l], 4 * QW))
        else:
            eprev, ch = _eres(eprev, ra, rb, _padb(g1e[l], HP), _padb(b1e[l], HP),
                              cp, _padb(bC[l], 4 * QW), write_e=(l < 2))
        at4 = at.reshape(4 * N, QW)
        bvt4 = bvt.reshape(4 * N, 2 * QW)
        write_r = l < 2
        ra, aa = _edge_sc(dst, src, at4, bvt4, ch, write_r, 0)
        rb, ab = _edge_sc(dst, src, at4, bvt4, ch, write_r, 1)
        h = _hupd(h, uh, aa, ab, _padb(g1h[l], HP), _padb(b1h[l], HP),
                  _pad2(W1[l], HP, FP), _padb(bf1[l], FP),
                  _pad2(W2[l], FP, HP), _padb(bf2[l], HP),
                  _padb(g2[l], HP), _padb(b2[l], HP))

    return _pool(h, batch)
